# Initial kernel scaffold; baseline (speedup 1.0000x reference)
import jax
import jax.numpy as jnp
from jax.experimental import pallas as pl


def kernel(x, edge_index1, edge_index2, W_self1, W_neigh1, b1, W_self2, W_neigh2, b2):
    raise NotImplementedError("write your pallas kernel here")


# speedup vs baseline: 3.8457x; 3.8457x over previous
import functools

import jax
import jax.numpy as jnp
from jax import lax
from jax.experimental import pallas as pl
from jax.experimental.pallas import tpu as pltpu
from jax.experimental.pallas import tpu_sc as plsc

_NC = 2
_NS = 16
_NW = _NC * _NS
_CH = 128
_BM = 400


def _sc_deg(n_rows_pad, e_pad):
    per_w = e_pad // _NW
    mesh = plsc.VectorSubcoreMesh(core_axis_name="c", subcore_axis_name="s")

    @functools.partial(
        pl.kernel,
        mesh=mesh,
        compiler_params=pltpu.CompilerParams(needs_layout_passes=False),
        out_type=(
            jax.ShapeDtypeStruct((_NW, n_rows_pad), jnp.float32),
            jax.ShapeDtypeStruct((_NW, n_rows_pad), jnp.float32),
        ),
        scratch_types=[
            pltpu.VMEM((per_w,), jnp.int32),
            pltpu.VMEM((n_rows_pad,), jnp.float32),
            pltpu.SemaphoreType.DMA,
        ],
    )
    def k(dst1_h, dst2_h, out1, out2, didx, hist, sem):
        c = lax.axis_index("c")
        s = lax.axis_index("s")
        wid = s * _NC + c
        ones16 = jnp.ones((16,), jnp.float32)

        def one_list(dst_h, out_h):
            def zbody(i, carry):
                hist[pl.ds(i * 16, 16)] = jnp.zeros((16,), jnp.float32)
                return carry

            lax.fori_loop(0, n_rows_pad // 16, zbody, 0)
            pltpu.sync_copy(dst_h.at[pl.ds(wid * per_w, per_w)], didx)

            def body(i, carry):
                iv = didx[pl.ds(i * 16, 16)]
                plsc.addupdate_scatter(hist, [iv], ones16)
                return carry

            lax.fori_loop(0, per_w // 16, body, 0)
            pltpu.sync_copy(hist, out_h.at[wid])

        one_list(dst1_h, out1)
        one_list(dst2_h, out2)

    return k


def _sc_acc(n_rows_pad, width, e_pad):
    per_w = e_pad // _NW
    n_ch = per_w // _CH
    rpt = n_rows_pad // _NS
    n_rch = rpt // _CH
    mesh = plsc.VectorSubcoreMesh(core_axis_name="c", subcore_axis_name="s")

    @functools.partial(
        pl.kernel,
        mesh=mesh,
        out_type=jax.ShapeDtypeStruct((_NC, n_rows_pad, width), jnp.float32),
        scratch_types=[
            pltpu.VMEM((_CH,), jnp.int32),
            pltpu.VMEM((_CH,), jnp.int32),
            pltpu.VMEM((_CH, width), jnp.float32),
            pltpu.VMEM_SHARED((n_rows_pad, width), jnp.float32),
            pltpu.SemaphoreType.DMA,
        ],
    )
    def k(feat, srcp, dstp, zw, out_acc, sidx, didx, rows, acc, sem):
        c = lax.axis_index("c")
        s = lax.axis_index("s")
        wid = s * _NC + c
        r0 = s * rpt
        pltpu.sync_copy(zw, rows)
        for j in range(n_rch):
            pltpu.sync_copy(rows, acc.at[pl.ds(r0 + j * _CH, _CH), :])
        plsc.subcore_barrier()
        base = wid * per_w

        def body(i, carry):
            off = base + i * _CH
            pltpu.sync_copy(srcp.at[pl.ds(off, _CH)], sidx)
            pltpu.sync_copy(dstp.at[pl.ds(off, _CH)], didx)
            pltpu.async_copy(feat.at[sidx], rows, sem).wait()
            pltpu.sync_copy(rows, acc.at[didx], add=True)
            return carry

        lax.fori_loop(0, n_ch, body, 0)
        plsc.subcore_barrier()
        for j in range(n_rch):
            rj = r0 + j * _CH
            pltpu.sync_copy(acc.at[pl.ds(rj, _CH), :], rows)
            pltpu.sync_copy(rows, out_acc.at[c, pl.ds(rj, _CH), :])

    return k


def _mm(x, w, bm):
    n, d = x.shape
    h = w.shape[1]

    def body(x_ref, w_ref, o_ref):
        o_ref[...] = jnp.dot(x_ref[...], w_ref[...],
                             preferred_element_type=jnp.float32)

    return pl.pallas_call(
        body,
        grid=(n // bm,),
        in_specs=[pl.BlockSpec((bm, d), lambda i: (i, 0)),
                  pl.BlockSpec((d, h), lambda i: (0, 0))],
        out_specs=pl.BlockSpec((bm, h), lambda i: (i, 0)),
        out_shape=jax.ShapeDtypeStruct((n, h), jnp.float32),
    )(x, w)


def _deg_reduce(deg1h, deg2h):
    nw, rp = deg1h.shape

    def body(d1, d2, o1, o2):
        ones_nw = jnp.ones((nw, 1), jnp.float32)
        dims = (((0,), (0,)), ((), ()))
        o1[...] = jnp.maximum(
            lax.dot_general(d1[...], ones_nw, dims,
                            preferred_element_type=jnp.float32), 1.0)
        o2[...] = jnp.maximum(
            lax.dot_general(d2[...], ones_nw, dims,
                            preferred_element_type=jnp.float32), 1.0)

    return pl.pallas_call(
        body,
        grid=(1,),
        in_specs=[pl.BlockSpec((nw, rp), lambda i: (0, 0)),
                  pl.BlockSpec((nw, rp), lambda i: (0, 0))],
        out_specs=[pl.BlockSpec((rp, 1), lambda i: (0, 0)),
                   pl.BlockSpec((rp, 1), lambda i: (0, 0))],
        out_shape=[jax.ShapeDtypeStruct((rp, 1), jnp.float32),
                   jax.ShapeDtypeStruct((rp, 1), jnp.float32)],
    )(deg1h, deg2h)


def _layer1_combine(x, acc_a, acc_b, deg1c, ws1, b1r, ws2, b2r):
    n, d = x.shape
    c = ws2.shape[1]

    def body(x_ref, aa, ab, dh, ws1_r, b1_r, ws2_r, b2_r, h_ref, hs2_ref):
        agg = aa[...] + ab[...]
        deg = dh[...]
        h = jnp.dot(x_ref[...], ws1_r[...],
                    preferred_element_type=jnp.float32)
        h = jnp.maximum(h + b1_r[...] + agg / deg, 0.0)
        h_ref[...] = h
        hs2_ref[...] = jnp.dot(h, ws2_r[...],
                               preferred_element_type=jnp.float32) + b2_r[...]

    return pl.pallas_call(
        body,
        grid=(n // _BM,),
        in_specs=[
            pl.BlockSpec((_BM, d), lambda i: (i, 0)),
            pl.BlockSpec((_BM, d), lambda i: (i, 0)),
            pl.BlockSpec((_BM, d), lambda i: (i, 0)),
            pl.BlockSpec((_BM, 1), lambda i: (i, 0)),
            pl.BlockSpec((d, d), lambda i: (0, 0)),
            pl.BlockSpec((1, d), lambda i: (0, 0)),
            pl.BlockSpec((d, c), lambda i: (0, 0)),
            pl.BlockSpec((1, c), lambda i: (0, 0)),
        ],
        out_specs=[pl.BlockSpec((_BM, d), lambda i: (i, 0)),
                   pl.BlockSpec((_BM, c), lambda i: (i, 0))],
        out_shape=[jax.ShapeDtypeStruct((n, d), jnp.float32),
                   jax.ShapeDtypeStruct((n, c), jnp.float32)],
    )(x, acc_a, acc_b, deg1c, ws1, b1r, ws2, b2r)


def _layer2_combine(hs2, acc_a, acc_b, deg2c, wn2):
    n, d = acc_a.shape
    c = wn2.shape[1]

    def body(hs_ref, aa, ab, dh, wn2_r, o_ref):
        agg = aa[...] + ab[...]
        deg = dh[...]
        o_ref[...] = hs_ref[...] + jnp.dot(
            agg / deg, wn2_r[...], preferred_element_type=jnp.float32)

    return pl.pallas_call(
        body,
        grid=(n // _BM,),
        in_specs=[
            pl.BlockSpec((_BM, c), lambda i: (i, 0)),
            pl.BlockSpec((_BM, d), lambda i: (i, 0)),
            pl.BlockSpec((_BM, d), lambda i: (i, 0)),
            pl.BlockSpec((_BM, 1), lambda i: (i, 0)),
            pl.BlockSpec((d, c), lambda i: (0, 0)),
        ],
        out_specs=pl.BlockSpec((_BM, c), lambda i: (i, 0)),
        out_shape=jax.ShapeDtypeStruct((n, c), jnp.float32),
    )(hs2, acc_a, acc_b, deg2c, wn2)


def kernel(x, edge_index1, edge_index2, W_self1, W_neigh1, b1,
           W_self2, W_neigh2, b2):
    n, d = x.shape
    h = W_neigh1.shape[1]
    c = W_neigh2.shape[1]
    e = edge_index1.shape[1]

    quant = _NW * _CH
    e_pad = ((e + quant - 1) // quant) * quant
    rq = _CH * _NS
    rp = ((n + 1 + rq - 1) // rq) * rq

    pad = e_pad - e
    pz = jnp.zeros((pad,), jnp.int32)
    pn = jnp.full((pad,), n, jnp.int32)
    src1 = jnp.concatenate([edge_index1[0], pz])
    dst1 = jnp.concatenate([edge_index1[1], pn])
    src2 = jnp.concatenate([edge_index2[0], pz])
    dst2 = jnp.concatenate([edge_index2[1], pn])

    z_h = jnp.zeros((_CH, h), jnp.float32)

    deg1h, deg2h = _sc_deg(rp, e_pad)(dst1, dst2)
    deg1c, deg2c = _deg_reduce(deg1h, deg2h)
    deg1c = deg1c[:n]
    deg2c = deg2c[:n]

    xw1 = _mm(x, W_neigh1, _BM)
    acc1 = _sc_acc(rp, h, e_pad)(xw1, src1, dst1, z_h)
    h_out, hs2 = _layer1_combine(
        x, acc1[0, :n], acc1[1, :n], deg1c,
        W_self1, b1.reshape(1, h), W_self2, b2.reshape(1, c))

    acc2 = _sc_acc(rp, h, e_pad)(h_out, src2, dst2, z_h)
    out = _layer2_combine(hs2, acc2[0, :n], acc2[1, :n], deg2c, W_neigh2)
    return out


# speedup vs baseline: 8.8463x; 2.3003x over previous
import functools

import jax
import jax.numpy as jnp
from jax import lax
from jax.experimental import pallas as pl
from jax.experimental.pallas import tpu as pltpu
from jax.experimental.pallas import tpu_sc as plsc

_NC = 2
_NS = 16
_NW = _NC * _NS
_CH = 128
_BM = 400


def _sc_deg(n_rows_pad, e_pad):
    per_w = e_pad // _NW
    mesh = plsc.VectorSubcoreMesh(core_axis_name="c", subcore_axis_name="s")

    @functools.partial(
        pl.kernel,
        mesh=mesh,
        compiler_params=pltpu.CompilerParams(needs_layout_passes=False),
        out_type=(
            jax.ShapeDtypeStruct((_NW, n_rows_pad), jnp.float32),
            jax.ShapeDtypeStruct((_NW, n_rows_pad), jnp.float32),
        ),
        scratch_types=[
            pltpu.VMEM((per_w,), jnp.int32),
            pltpu.VMEM((n_rows_pad,), jnp.float32),
            pltpu.SemaphoreType.DMA,
        ],
    )
    def k(dst1_h, dst2_h, out1, out2, didx, hist, sem):
        c = lax.axis_index("c")
        s = lax.axis_index("s")
        wid = s * _NC + c
        ones16 = jnp.ones((16,), jnp.float32)

        def one_list(dst_h, out_h):
            def zbody(i, carry):
                hist[pl.ds(i * 16, 16)] = jnp.zeros((16,), jnp.float32)
                return carry

            lax.fori_loop(0, n_rows_pad // 16, zbody, 0)
            pltpu.sync_copy(dst_h.at[pl.ds(wid * per_w, per_w)], didx)

            def body(i, carry):
                iv = didx[pl.ds(i * 16, 16)]
                plsc.addupdate_scatter(hist, [iv], ones16)
                return carry

            lax.fori_loop(0, per_w // 16, body, 0)
            pltpu.sync_copy(hist, out_h.at[wid])

        one_list(dst1_h, out1)
        one_list(dst2_h, out2)

    return k


def _sc_acc(n_rows_pad, width, e_pad):
    per_w = e_pad // _NW
    n_ch = per_w // _CH
    n_pair = n_ch // 2
    rpt = n_rows_pad // _NS
    n_rch = rpt // _CH
    mesh = plsc.VectorSubcoreMesh(core_axis_name="c", subcore_axis_name="s")

    @functools.partial(
        pl.kernel,
        mesh=mesh,
        out_type=jax.ShapeDtypeStruct((_NC, n_rows_pad, width), jnp.float32),
        scratch_types=[
            pltpu.VMEM((_CH,), jnp.int32),
            pltpu.VMEM((_CH,), jnp.int32),
            pltpu.VMEM((n_ch, _CH), jnp.int32),
            pltpu.VMEM((_CH, width), jnp.float32),
            pltpu.VMEM((_CH, width), jnp.float32),
            pltpu.VMEM_SHARED((n_rows_pad, width), jnp.float32),
            pltpu.SemaphoreType.DMA,
            pltpu.SemaphoreType.DMA,
            pltpu.SemaphoreType.DMA,
        ],
    )
    def k(feat, srcp, dstp, zw, out_acc, sidx0, sidx1, didx, rows0, rows1,
          acc, sem0, sem1, semi):
        c = lax.axis_index("c")
        s = lax.axis_index("s")
        wid = s * _NC + c
        r0 = s * rpt
        row0 = wid * n_ch
        pltpu.sync_copy(dstp.at[pl.ds(row0, n_ch), :], didx)
        pltpu.sync_copy(zw, rows0)
        for j in range(n_rch):
            pltpu.sync_copy(rows0, acc.at[pl.ds(r0 + j * _CH, _CH), :])
        plsc.subcore_barrier()

        pltpu.sync_copy(srcp.at[row0], sidx0)
        pltpu.async_copy(feat.at[sidx0], rows0, sem0)

        def body(p, carry):
            i = 2 * p
            pltpu.async_copy(srcp.at[row0 + i + 1], sidx1, semi)
            pltpu.make_async_copy(feat.at[sidx0], rows0, sem0).wait()
            pltpu.make_async_copy(srcp.at[row0 + i + 1], sidx1,
                                  semi).wait()
            pltpu.async_copy(feat.at[sidx1], rows1, sem1)
            pltpu.sync_copy(rows0, acc.at[didx.at[i]], add=True)

            @pl.when(p < n_pair - 1)
            def _():
                pltpu.async_copy(srcp.at[row0 + i + 2], sidx0, semi)

            pltpu.make_async_copy(feat.at[sidx1], rows1, sem1).wait()

            @pl.when(p < n_pair - 1)
            def _():
                pltpu.make_async_copy(srcp.at[row0 + i + 2], sidx0,
                                      semi).wait()
                pltpu.async_copy(feat.at[sidx0], rows0, sem0)

            pltpu.sync_copy(rows1, acc.at[didx.at[i + 1]], add=True)
            return carry

        lax.fori_loop(0, n_pair, body, 0)
        plsc.subcore_barrier()
        for j in range(n_rch):
            rj = r0 + j * _CH
            pltpu.sync_copy(acc.at[pl.ds(rj, _CH), :], rows0)
            pltpu.sync_copy(rows0, out_acc.at[c, pl.ds(rj, _CH), :])

    return k


def _mm(x, w, bm):
    n, d = x.shape
    h = w.shape[1]

    def body(x_ref, w_ref, o_ref):
        o_ref[...] = jnp.dot(x_ref[...], w_ref[...],
                             preferred_element_type=jnp.float32)

    return pl.pallas_call(
        body,
        grid=(n // bm,),
        in_specs=[pl.BlockSpec((bm, d), lambda i: (i, 0)),
                  pl.BlockSpec((d, h), lambda i: (0, 0))],
        out_specs=pl.BlockSpec((bm, h), lambda i: (i, 0)),
        out_shape=jax.ShapeDtypeStruct((n, h), jnp.float32),
    )(x, w)


def _deg_reduce(deg1h, deg2h):
    nw, rp = deg1h.shape

    def body(d1, d2, o1, o2):
        ones_nw = jnp.ones((nw, 1), jnp.float32)
        dims = (((0,), (0,)), ((), ()))
        o1[...] = jnp.maximum(
            lax.dot_general(d1[...], ones_nw, dims,
                            preferred_element_type=jnp.float32), 1.0)
        o2[...] = jnp.maximum(
            lax.dot_general(d2[...], ones_nw, dims,
                            preferred_element_type=jnp.float32), 1.0)

    return pl.pallas_call(
        body,
        grid=(1,),
        in_specs=[pl.BlockSpec((nw, rp), lambda i: (0, 0)),
                  pl.BlockSpec((nw, rp), lambda i: (0, 0))],
        out_specs=[pl.BlockSpec((rp, 1), lambda i: (0, 0)),
                   pl.BlockSpec((rp, 1), lambda i: (0, 0))],
        out_shape=[jax.ShapeDtypeStruct((rp, 1), jnp.float32),
                   jax.ShapeDtypeStruct((rp, 1), jnp.float32)],
    )(deg1h, deg2h)


def _layer1_combine(x, acc_a, acc_b, deg1c, ws1, b1r, ws2, b2r):
    n, d = x.shape
    c = ws2.shape[1]

    def body(x_ref, aa, ab, dh, ws1_r, b1_r, ws2_r, b2_r, h_ref, hs2_ref):
        agg = aa[...] + ab[...]
        deg = dh[...]
        h = jnp.dot(x_ref[...], ws1_r[...],
                    preferred_element_type=jnp.float32)
        h = jnp.maximum(h + b1_r[...] + agg / deg, 0.0)
        h_ref[...] = h
        hs2_ref[...] = jnp.dot(h, ws2_r[...],
                               preferred_element_type=jnp.float32) + b2_r[...]

    return pl.pallas_call(
        body,
        grid=(n // _BM,),
        in_specs=[
            pl.BlockSpec((_BM, d), lambda i: (i, 0)),
            pl.BlockSpec((_BM, d), lambda i: (i, 0)),
            pl.BlockSpec((_BM, d), lambda i: (i, 0)),
            pl.BlockSpec((_BM, 1), lambda i: (i, 0)),
            pl.BlockSpec((d, d), lambda i: (0, 0)),
            pl.BlockSpec((1, d), lambda i: (0, 0)),
            pl.BlockSpec((d, c), lambda i: (0, 0)),
            pl.BlockSpec((1, c), lambda i: (0, 0)),
        ],
        out_specs=[pl.BlockSpec((_BM, d), lambda i: (i, 0)),
                   pl.BlockSpec((_BM, c), lambda i: (i, 0))],
        out_shape=[jax.ShapeDtypeStruct((n, d), jnp.float32),
                   jax.ShapeDtypeStruct((n, c), jnp.float32)],
    )(x, acc_a, acc_b, deg1c, ws1, b1r, ws2, b2r)


def _layer2_combine(hs2, acc_a, acc_b, deg2c, wn2):
    n, d = acc_a.shape
    c = wn2.shape[1]

    def body(hs_ref, aa, ab, dh, wn2_r, o_ref):
        agg = aa[...] + ab[...]
        deg = dh[...]
        o_ref[...] = hs_ref[...] + jnp.dot(
            agg / deg, wn2_r[...], preferred_element_type=jnp.float32)

    return pl.pallas_call(
        body,
        grid=(n // _BM,),
        in_specs=[
            pl.BlockSpec((_BM, c), lambda i: (i, 0)),
            pl.BlockSpec((_BM, d), lambda i: (i, 0)),
            pl.BlockSpec((_BM, d), lambda i: (i, 0)),
            pl.BlockSpec((_BM, 1), lambda i: (i, 0)),
            pl.BlockSpec((d, c), lambda i: (0, 0)),
        ],
        out_specs=pl.BlockSpec((_BM, c), lambda i: (i, 0)),
        out_shape=jax.ShapeDtypeStruct((n, c), jnp.float32),
    )(hs2, acc_a, acc_b, deg2c, wn2)


def kernel(x, edge_index1, edge_index2, W_self1, W_neigh1, b1,
           W_self2, W_neigh2, b2):
    n, d = x.shape
    h = W_neigh1.shape[1]
    c = W_neigh2.shape[1]
    e = edge_index1.shape[1]

    quant = _NW * _CH * 2
    e_pad = ((e + quant - 1) // quant) * quant
    rq = _CH * _NS
    rp = ((n + 1 + rq - 1) // rq) * rq

    pad = e_pad - e
    pz = (jnp.arange(pad, dtype=jnp.int32) % n)
    pn = n + (jnp.arange(pad, dtype=jnp.int32) % (rp - n))
    src1f = jnp.concatenate([edge_index1[0], pz])
    dst1f = jnp.concatenate([edge_index1[1], pn])
    src2f = jnp.concatenate([edge_index2[0], pz])
    dst2f = jnp.concatenate([edge_index2[1], pn])
    src1 = src1f.reshape(-1, _CH)
    dst1 = dst1f.reshape(-1, _CH)
    src2 = src2f.reshape(-1, _CH)
    dst2 = dst2f.reshape(-1, _CH)

    z_h = jnp.zeros((_CH, h), jnp.float32)

    deg1h, deg2h = _sc_deg(rp, e_pad)(dst1f, dst2f)
    deg1c, deg2c = _deg_reduce(deg1h, deg2h)
    deg1c = deg1c[:n]
    deg2c = deg2c[:n]

    xw1 = _mm(x, W_neigh1, _BM)
    acc1 = _sc_acc(rp, h, e_pad)(xw1, src1, dst1, z_h)
    h_out, hs2 = _layer1_combine(
        x, acc1[0, :n], acc1[1, :n], deg1c,
        W_self1, b1.reshape(1, h), W_self2, b2.reshape(1, c))

    acc2 = _sc_acc(rp, h, e_pad)(h_out, src2, dst2, z_h)
    out = _layer2_combine(hs2, acc2[0, :n], acc2[1, :n], deg2c, W_neigh2)
    return out
